# Initial kernel scaffold; baseline (speedup 1.0000x reference)
import functools

import jax
import jax.numpy as jnp
from jax import lax
from jax.experimental import pallas as pl
from jax.experimental.pallas import tpu as pltpu
from jax.experimental.pallas import tpu_sc as plsc

CLIP_LIMIT = 2.0
TILES_Y, TILES_X = 8, 8
L = 16
NC, NS = 2, 16
NW = NC * NS


@functools.lru_cache(maxsize=None)
def _build(B, H, W):
    th = H // TILES_Y
    tw = W // TILES_X
    n_tiles = B * TILES_Y * TILES_X
    tpw = n_tiles // NW
    assert n_tiles % NW == 0 and tw % L == 0
    nbins = 256
    clip_count = float(th * tw) * CLIP_LIMIT / nbins

    mesh = plsc.VectorSubcoreMesh(
        core_axis_name="c", subcore_axis_name="s", num_cores=NC, num_subcores=NS
    )

    @functools.partial(
        pl.kernel,
        out_type=jax.ShapeDtypeStruct((B, H, W), jnp.float32),
        mesh=mesh,
        scratch_types=[
            pltpu.VMEM((th, tw), jnp.float32),
            pltpu.VMEM((th, tw), jnp.float32),
            pltpu.VMEM((nbins * L,), jnp.float32),
            pltpu.VMEM((nbins,), jnp.float32),
            pltpu.VMEM((nbins,), jnp.float32),
            pltpu.VMEM((L,), jnp.float32),
        ],
    )
    def clahe_kernel(img_hbm, out_hbm, tile_in, tile_out, hist, clipped, table, exc):
        cid = lax.axis_index("c")
        sid = lax.axis_index("s")
        wid = sid * NC + cid
        lanes = lax.iota(jnp.int32, L)
        zeros16 = jnp.zeros((L,), jnp.float32)
        ones16 = jnp.ones((L,), jnp.float32)

        @pl.loop(0, nbins)
        def _zero(i):
            hist[pl.ds(i * L, L)] = zeros16

        exc[...] = zeros16

        @pl.loop(0, tpw)
        def _tile(j):
            t = wid * tpw + j
            b = t // (TILES_Y * TILES_X)
            rem = t % (TILES_Y * TILES_X)
            ty = rem // TILES_X
            tx = rem % TILES_X
            ry = pl.ds(ty * th, th)
            rx = pl.ds(tx * tw, tw)
            pltpu.sync_copy(img_hbm.at[b, ry, rx], tile_in)

            @pl.loop(0, th)
            def _row(r):
                for cc in range(tw // L):
                    v = tile_in[r, pl.ds(cc * L, L)]
                    bi = jnp.clip((v * 256.0).astype(jnp.int32), 0, 255)
                    plsc.addupdate_scatter(hist, [bi * L + lanes], ones16)

            @pl.loop(0, nbins // L)
            def _reduce(c):
                hch = zeros16
                for i in range(L):
                    base = (c * L + i) * L
                    row = hist[pl.ds(base, L)]
                    hist[pl.ds(base, L)] = zeros16
                    s = jnp.sum(row)
                    hch = jnp.where(lanes == i, s, hch)
                clipped[pl.ds(c * L, L)] = jnp.minimum(hch, clip_count)
                exc[...] = exc[...] + jnp.maximum(hch - clip_count, 0.0)

            excess = jnp.sum(exc[...])
            exc[...] = zeros16
            add_per_bin = excess / float(nbins)

            def _cdf(c, run):
                v = clipped[pl.ds(c * L, L)] + add_per_bin
                table[pl.ds(c * L, L)] = plsc.cumsum(v) + run
                return run + jnp.sum(v)

            total = lax.fori_loop(0, nbins // L, _cdf, jnp.float32(0.0))

            @pl.loop(0, nbins // L)
            def _norm(c):
                table[pl.ds(c * L, L)] = table[pl.ds(c * L, L)] / total

            @pl.loop(0, th)
            def _lookup(r):
                for cc in range(tw // L):
                    v = tile_in[r, pl.ds(cc * L, L)]
                    ii = jnp.clip((v * 255.0).astype(jnp.int32), 0, 255)
                    tile_out[r, pl.ds(cc * L, L)] = plsc.load_gather(table, [ii])

            pltpu.sync_copy(tile_out, out_hbm.at[b, ry, rx])

    return clahe_kernel


@jax.jit
def kernel(image):
    B, C, H, W = image.shape
    out = _build(B, H, W)(image[:, 0])
    return out[:, None]


# speedup vs baseline: 206.0883x; 206.0883x over previous
import functools

import jax
import jax.numpy as jnp
from jax import lax
from jax.experimental import pallas as pl
from jax.experimental.pallas import tpu as pltpu
from jax.experimental.pallas import tpu_sc as plsc

CLIP_LIMIT = 2.0
TILES_Y, TILES_X = 8, 8
L = 16
NC, NS = 2, 16
NW = NC * NS


@functools.lru_cache(maxsize=None)
def _build(B, H, W):
    th = H // TILES_Y
    tw = W // TILES_X
    n_tiles = B * TILES_Y * TILES_X
    tpw = n_tiles // NW
    assert n_tiles % NW == 0 and tw % L == 0
    nbins = 256
    clip_count = float(th * tw) * CLIP_LIMIT / nbins

    mesh = plsc.VectorSubcoreMesh(
        core_axis_name="c", subcore_axis_name="s", num_cores=NC, num_subcores=NS
    )

    @functools.partial(
        pl.kernel,
        out_type=jax.ShapeDtypeStruct((B, H, W), jnp.float32),
        mesh=mesh,
        compiler_params=pltpu.CompilerParams(
            use_tc_tiling_on_sc=False, needs_layout_passes=False
        ),
        scratch_types=[
            pltpu.VMEM((th, tw), jnp.float32),
            pltpu.VMEM((th, tw), jnp.float32),
            pltpu.VMEM((nbins * L,), jnp.float32),
            pltpu.VMEM((nbins,), jnp.float32),
            pltpu.VMEM((nbins,), jnp.float32),
            pltpu.VMEM((L,), jnp.float32),
        ],
    )
    def clahe_kernel(img_hbm, out_hbm, tile_in, tile_out, hist, clipped, table, exc):
        cid = lax.axis_index("c")
        sid = lax.axis_index("s")
        wid = sid * NC + cid
        lanes = lax.iota(jnp.int32, L)
        zeros16 = jnp.zeros((L,), jnp.float32)
        ones16 = jnp.ones((L,), jnp.float32)

        @pl.loop(0, nbins)
        def _zero(i):
            hist[pl.ds(i * L, L)] = zeros16

        exc[...] = zeros16

        @pl.loop(0, tpw)
        def _tile(j):
            t = wid * tpw + j
            b = t // (TILES_Y * TILES_X)
            rem = t % (TILES_Y * TILES_X)
            ty = rem // TILES_X
            tx = rem % TILES_X
            ry = pl.ds(ty * th, th)
            rx = pl.ds(tx * tw, tw)
            pltpu.sync_copy(img_hbm.at[b, ry, rx], tile_in)

            @pl.loop(0, th)
            def _row(r):
                for cc in range(tw // L):
                    v = tile_in[r, pl.ds(cc * L, L)]
                    bi = jnp.clip((v * 256.0).astype(jnp.int32), 0, 255)
                    plsc.addupdate_scatter(hist, [bi * L + lanes], ones16)

            @pl.loop(0, nbins // L)
            def _reduce(c):
                hch = zeros16
                for i in range(L):
                    base = (c * L + i) * L
                    row = hist[pl.ds(base, L)]
                    hist[pl.ds(base, L)] = zeros16
                    s = jnp.sum(row)
                    hch = jnp.where(lanes == i, s, hch)
                clipped[pl.ds(c * L, L)] = jnp.minimum(hch, clip_count)
                exc[...] = exc[...] + jnp.maximum(hch - clip_count, 0.0)

            excess = jnp.sum(exc[...])
            exc[...] = zeros16
            add_per_bin = excess * (1.0 / float(nbins))

            def _cdf(c, run):
                v = clipped[pl.ds(c * L, L)] + add_per_bin
                table[pl.ds(c * L, L)] = plsc.cumsum(v) + run
                return run + jnp.sum(v)

            total = lax.fori_loop(0, nbins // L, _cdf, jnp.float32(0.0))

            @pl.loop(0, nbins // L)
            def _norm(c):
                table[pl.ds(c * L, L)] = table[pl.ds(c * L, L)] / total

            @pl.loop(0, th)
            def _lookup(r):
                for cc in range(tw // L):
                    v = tile_in[r, pl.ds(cc * L, L)]
                    ii = jnp.clip((v * 255.0).astype(jnp.int32), 0, 255)
                    tile_out[r, pl.ds(cc * L, L)] = plsc.load_gather(table, [ii])

            pltpu.sync_copy(tile_out, out_hbm.at[b, ry, rx])

    return clahe_kernel


@jax.jit
def kernel(image):
    B, C, H, W = image.shape
    out = _build(B, H, W)(image[:, 0])
    return out[:, None]
